# baseline (speedup 1.0000x reference)
import jax
import jax.numpy as jnp
from jax import lax
from jax.experimental import pallas as pl
from jax.experimental.pallas import tpu as pltpu
from jax.experimental.pallas import tpu_sc as plsc

B = 16384
F_SPARSE = 26
EMB = 16
VOCAB = 100000
DENSE = 13
F_PAD = 32
SLABS = 4
FPS = 8

NC = 2
NS = 16
NW = NC * NS
N_ROWS = B * F_PAD
PER_W = N_ROWS // NW
CHUNK = 512
N_CHUNKS = PER_W // CHUNK
OUT_R = N_ROWS * EMB // 128
OROWS = CHUNK * EMB // 128

VBLK = 1024
NVB = 98
VOCAB_P = NVB * VBLK
TROWS = VOCAB_P * EMB // 128
FLAT_R = F_SPARSE * TROWS


def _tr_body(t_ref, o_ref):
    x = t_ref[0]
    o_ref[...] = x.T.reshape(VBLK * EMB // 128, 128)


def _tc_detile(tabt, interpret=False):
    return pl.pallas_call(
        _tr_body,
        grid=(F_SPARSE, NVB),
        in_specs=[pl.BlockSpec((1, EMB, VBLK), lambda f, c: (f, 0, c))],
        out_specs=pl.BlockSpec((VBLK * EMB // 128, 128),
                               lambda f, c: (f * NVB + c, 0)),
        out_shape=jax.ShapeDtypeStruct((FLAT_R, 128), jnp.float32),
        interpret=interpret,
    )(tabt)


def _repack(rows_v, o_v):
    def rep(d, _):
        for m in range(FPS):
            o_v[d, pl.ds(EMB * m, EMB)] = rows_v[d * FPS + m, :]
        return 0
    lax.fori_loop(0, OROWS, rep, 0)


def _sc_gather_body(idx_hbm, tab_hbm, out_hbm, idx_v, rows0, rows1,
                    o_v0, o_v1, sem0, sem1, osem0, osem1):
    wid = lax.axis_index("s") * NC + lax.axis_index("c")
    base = wid * PER_W
    pltpu.sync_copy(idx_hbm.at[wid], idx_v)

    def out_slice(j):
        start = pl.multiple_of((base + j * CHUNK) * EMB // 128, OROWS)
        return out_hbm.at[pl.ds(start, OROWS)]

    pltpu.async_copy(tab_hbm.at[idx_v.at[0]], rows0, sem0)

    def pair(jj, _):
        j0 = 2 * jj
        pltpu.async_copy(tab_hbm.at[idx_v.at[j0 + 1]], rows1, sem1)
        pltpu.make_async_copy(tab_hbm.at[idx_v.at[j0]], rows0, sem0).wait()

        @pl.when(jj > 0)
        def _():
            pltpu.make_async_copy(o_v0, out_slice(j0), osem0).wait()
        _repack(rows0, o_v0)
        pltpu.async_copy(o_v0, out_slice(j0), osem0)

        jn = jnp.minimum(j0 + 2, N_CHUNKS - 1)
        pltpu.async_copy(tab_hbm.at[idx_v.at[jn]], rows0, sem0)
        pltpu.make_async_copy(tab_hbm.at[idx_v.at[j0 + 1]], rows1, sem1).wait()

        @pl.when(jj > 0)
        def _():
            pltpu.make_async_copy(o_v1, out_slice(j0 + 1), osem1).wait()
        _repack(rows1, o_v1)
        pltpu.async_copy(o_v1, out_slice(j0 + 1), osem1)
        return 0

    lax.fori_loop(0, N_CHUNKS // 2, pair, 0)
    pltpu.make_async_copy(
        tab_hbm.at[idx_v.at[N_CHUNKS - 1]], rows0, sem0).wait()
    pltpu.make_async_copy(o_v0, out_slice(N_CHUNKS - 2), osem0).wait()
    pltpu.make_async_copy(o_v1, out_slice(N_CHUNKS - 1), osem1).wait()


def _sc_gather(idx, tab_flat, interpret=False):
    mesh = plsc.VectorSubcoreMesh(
        core_axis_name="c", subcore_axis_name="s",
        num_cores=NC, num_subcores=NS)
    return pl.kernel(
        _sc_gather_body,
        out_type=jax.ShapeDtypeStruct((OUT_R, 128), jnp.float32),
        mesh=mesh,
        scratch_types=[
            pltpu.VMEM((N_CHUNKS, CHUNK), jnp.int32),
            pltpu.VMEM((CHUNK, EMB), jnp.float32),
            pltpu.VMEM((CHUNK, EMB), jnp.float32),
            pltpu.VMEM((OROWS, 128), jnp.float32),
            pltpu.VMEM((OROWS, 128), jnp.float32),
            pltpu.SemaphoreType.DMA,
            pltpu.SemaphoreType.DMA,
            pltpu.SemaphoreType.DMA,
            pltpu.SemaphoreType.DMA,
        ],
        compiler_params=pltpu.CompilerParams(use_tc_tiling_on_sc=False),
        interpret=interpret,
    )(idx, tab_flat)


def _mlp_body(g_ref, xd_ref, w1p_ref, w1d_ref, b1_ref, w2_ref, b2_ref,
              w3_ref, b3_ref, ww_ref, bw_ref, mix_ref,
              logit_ref, prob_ref):
    xd = xd_ref[...]
    h = jnp.dot(xd, w1d_ref[...], preferred_element_type=jnp.float32)
    for r in range(SLABS):
        h += jnp.dot(g_ref[r], w1p_ref[128 * r:128 * (r + 1), :],
                     preferred_element_type=jnp.float32)
    h = jnp.maximum(h + b1_ref[...], 0.0)
    h = jnp.maximum(
        jnp.dot(h, w2_ref[...], preferred_element_type=jnp.float32)
        + b2_ref[...], 0.0)
    deep = jnp.dot(h, w3_ref[...], preferred_element_type=jnp.float32) + b3_ref[...]
    wide = jnp.dot(xd, ww_ref[...], preferred_element_type=jnp.float32) + bw_ref[...]
    e = jnp.exp(mix_ref[...] - jnp.max(mix_ref[...]))
    w = e / jnp.sum(e)
    logit = wide * w[0:1, 0:1] + deep * w[0:1, 1:2]
    logit_ref[...] = logit
    prob_ref[...] = 1.0 / (1.0 + jnp.exp(-logit))


def _mlp(g, xd, w1p, w1d, b1, w2, b2, w3, b3, ww, bw, mix, interpret=False):
    BB = 2048
    grid = (B // BB,)
    const = lambda shape: pl.BlockSpec(shape, lambda i: tuple(0 for _ in shape))
    return pl.pallas_call(
        _mlp_body,
        grid=grid,
        in_specs=[
            pl.BlockSpec((SLABS, BB, 128), lambda i: (0, i, 0)),
            pl.BlockSpec((BB, DENSE), lambda i: (i, 0)),
            const((SLABS * 128, 64)),
            const((DENSE, 64)),
            const((1, 64)),
            const((64, 32)),
            const((1, 32)),
            const((32, 1)),
            const((1, 1)),
            const((DENSE, 1)),
            const((1, 1)),
            const((1, 2)),
        ],
        out_specs=[
            pl.BlockSpec((BB, 1), lambda i: (i, 0)),
            pl.BlockSpec((BB, 1), lambda i: (i, 0)),
        ],
        out_shape=[
            jax.ShapeDtypeStruct((B, 1), jnp.float32),
            jax.ShapeDtypeStruct((B, 1), jnp.float32),
        ],
        interpret=interpret,
    )(g, xd, w1p, w1d, b1, w2, b2, w3, b3, ww, bw, mix)


@jax.jit
def kernel(x_sparse, x_dense, tables, W_wide, b_wide, W1, b1, W2, b2, W3, b3, mix):
    tab_flat = tables.reshape(F_SPARSE * VOCAB, EMB)
    offs = (jnp.arange(F_SPARSE, dtype=jnp.int32) * VOCAB)[None, :]
    idx_pad = jnp.concatenate(
        [x_sparse.astype(jnp.int32) + offs,
         jnp.zeros((B, F_PAD - F_SPARSE), jnp.int32)], axis=1)
    idx_sm = idx_pad.reshape(B, SLABS, FPS).transpose(1, 0, 2)
    idx = idx_sm.reshape(NW, N_CHUNKS, CHUNK)
    g = _sc_gather(idx, tab_flat).reshape(SLABS, B, 128)
    w1p = jnp.concatenate(
        [W1[:F_SPARSE * EMB], jnp.zeros((SLABS * 128 - F_SPARSE * EMB, 64),
                                        jnp.float32)], axis=0)
    logit, prob = _mlp(
        g, x_dense,
        w1p, W1[F_SPARSE * EMB:], b1.reshape(1, 64),
        W2, b2.reshape(1, 32), W3, b3.reshape(1, 1),
        W_wide, b_wide.reshape(1, 1), mix.reshape(1, 2))
    return (logit, prob)


# speedup vs baseline: 1.1625x; 1.1625x over previous
import jax
import jax.numpy as jnp
from jax import lax
from jax.experimental import pallas as pl
from jax.experimental.pallas import tpu as pltpu
from jax.experimental.pallas import tpu_sc as plsc

B = 16384
F_SPARSE = 26
EMB = 16
VOCAB = 100000
DENSE = 13
SE_DIM = F_SPARSE * EMB

NC = 2
NS = 16
NW = NC * NS
N_ROWS = B * F_SPARSE
PER_W = N_ROWS // NW
CHUNK = 512
N_CHUNKS = PER_W // CHUNK


def _sc_gather_body(idx_hbm, tab_hbm, out_hbm, idx_v, rows0, rows1,
                    sem0, sem1, osem0, osem1):
    wid = lax.axis_index("s") * NC + lax.axis_index("c")
    base = wid * PER_W
    pltpu.sync_copy(idx_hbm.at[wid], idx_v)

    def out_slice(j):
        start = pl.multiple_of(base + j * CHUNK, CHUNK)
        return out_hbm.at[pl.ds(start, CHUNK)]

    pltpu.async_copy(tab_hbm.at[idx_v.at[0]], rows0, sem0)

    def pair(jj, _):
        j0 = 2 * jj

        @pl.when(jj > 0)
        def _():
            pltpu.make_async_copy(rows1, out_slice(j0 - 1), osem1).wait()
        pltpu.async_copy(tab_hbm.at[idx_v.at[j0 + 1]], rows1, sem1)
        pltpu.make_async_copy(tab_hbm.at[idx_v.at[j0]], rows0, sem0).wait()
        pltpu.async_copy(rows0, out_slice(j0), osem0)

        pltpu.make_async_copy(rows0, out_slice(j0), osem0).wait()
        jn = jnp.minimum(j0 + 2, N_CHUNKS - 1)
        pltpu.async_copy(tab_hbm.at[idx_v.at[jn]], rows0, sem0)
        pltpu.make_async_copy(tab_hbm.at[idx_v.at[j0 + 1]], rows1, sem1).wait()
        pltpu.async_copy(rows1, out_slice(j0 + 1), osem1)
        return 0

    lax.fori_loop(0, N_CHUNKS // 2, pair, 0)
    pltpu.make_async_copy(
        tab_hbm.at[idx_v.at[N_CHUNKS - 1]], rows0, sem0).wait()
    pltpu.make_async_copy(rows1, out_slice(N_CHUNKS - 1), osem1).wait()


def _sc_gather(idx, tab_flat, interpret=False):
    mesh = plsc.VectorSubcoreMesh(
        core_axis_name="c", subcore_axis_name="s",
        num_cores=NC, num_subcores=NS)
    return pl.kernel(
        _sc_gather_body,
        out_type=jax.ShapeDtypeStruct((N_ROWS, EMB), jnp.bfloat16),
        mesh=mesh,
        scratch_types=[
            pltpu.VMEM((N_CHUNKS, CHUNK), jnp.int32),
            pltpu.VMEM((CHUNK, EMB), jnp.bfloat16),
            pltpu.VMEM((CHUNK, EMB), jnp.bfloat16),
            pltpu.SemaphoreType.DMA,
            pltpu.SemaphoreType.DMA,
            pltpu.SemaphoreType.DMA,
            pltpu.SemaphoreType.DMA,
        ],
        compiler_params=pltpu.CompilerParams(use_tc_tiling_on_sc=False),
        interpret=interpret,
    )(idx, tab_flat)


def _mlp_body(se_ref, xd_ref, w1s_ref, w1d_ref, b1_ref, w2_ref, b2_ref,
              w3_ref, b3_ref, ww_ref, bw_ref, mix_ref,
              logit_ref, prob_ref):
    xd = xd_ref[...]
    h = jnp.dot(se_ref[...], w1s_ref[...], preferred_element_type=jnp.float32)
    h += jnp.dot(xd, w1d_ref[...], preferred_element_type=jnp.float32)
    h = jnp.maximum(h + b1_ref[...], 0.0)
    h = jnp.maximum(
        jnp.dot(h, w2_ref[...], preferred_element_type=jnp.float32)
        + b2_ref[...], 0.0)
    deep = jnp.dot(h, w3_ref[...], preferred_element_type=jnp.float32) + b3_ref[...]
    wide = jnp.dot(xd, ww_ref[...], preferred_element_type=jnp.float32) + bw_ref[...]
    e = jnp.exp(mix_ref[...] - jnp.max(mix_ref[...]))
    w = e / jnp.sum(e)
    logit = wide * w[0:1, 0:1] + deep * w[0:1, 1:2]
    logit_ref[...] = logit
    prob_ref[...] = 1.0 / (1.0 + jnp.exp(-logit))


def _mlp(se, xd, w1s, w1d, b1, w2, b2, w3, b3, ww, bw, mix, interpret=False):
    BB = 2048
    grid = (B // BB,)
    const = lambda shape: pl.BlockSpec(shape, lambda i: tuple(0 for _ in shape))
    return pl.pallas_call(
        _mlp_body,
        grid=grid,
        in_specs=[
            pl.BlockSpec((BB, SE_DIM), lambda i: (i, 0)),
            pl.BlockSpec((BB, DENSE), lambda i: (i, 0)),
            const((SE_DIM, 64)),
            const((DENSE, 64)),
            const((1, 64)),
            const((64, 32)),
            const((1, 32)),
            const((32, 1)),
            const((1, 1)),
            const((DENSE, 1)),
            const((1, 1)),
            const((1, 2)),
        ],
        out_specs=[
            pl.BlockSpec((BB, 1), lambda i: (i, 0)),
            pl.BlockSpec((BB, 1), lambda i: (i, 0)),
        ],
        out_shape=[
            jax.ShapeDtypeStruct((B, 1), jnp.float32),
            jax.ShapeDtypeStruct((B, 1), jnp.float32),
        ],
        interpret=interpret,
    )(se, xd, w1s, w1d, b1, w2, b2, w3, b3, ww, bw, mix)


@jax.jit
def kernel(x_sparse, x_dense, tables, W_wide, b_wide, W1, b1, W2, b2, W3, b3, mix):
    tab_flat = tables.astype(jnp.bfloat16).reshape(F_SPARSE * VOCAB, EMB)
    offs = (jnp.arange(F_SPARSE, dtype=jnp.int32) * VOCAB)[None, :]
    idx = (x_sparse.astype(jnp.int32) + offs).reshape(NW, N_CHUNKS, CHUNK)
    se = _sc_gather(idx, tab_flat).reshape(B, SE_DIM)
    logit, prob = _mlp(
        se, x_dense,
        W1[:SE_DIM].astype(jnp.bfloat16), W1[SE_DIM:], b1.reshape(1, 64),
        W2, b2.reshape(1, 32), W3, b3.reshape(1, 1),
        W_wide, b_wide.reshape(1, 1), mix.reshape(1, 2))
    return (logit, prob)


# speedup vs baseline: 1.3948x; 1.1998x over previous
import jax
import jax.numpy as jnp
from jax import lax
from jax.experimental import pallas as pl
from jax.experimental.pallas import tpu as pltpu
from jax.experimental.pallas import tpu_sc as plsc

B = 16384
F_SPARSE = 26
EMB = 16
VOCAB = 100000
DENSE = 13
SE_DIM = F_SPARSE * EMB

NC = 2
NS = 16
NW = NC * NS
N_ROWS = B * F_SPARSE
PER_W = N_ROWS // NW
CHUNK = 512
N_CHUNKS = PER_W // CHUNK


def _sc_gather_body(idx_hbm, tab_hbm, out_hbm, idx_v, rows0, rows1,
                    sem0, sem1, osem0, osem1):
    wid = lax.axis_index("s") * NC + lax.axis_index("c")
    base = wid * PER_W
    pltpu.sync_copy(idx_hbm.at[wid], idx_v)

    def out_slice(j):
        start = pl.multiple_of(base + j * CHUNK, CHUNK)
        return out_hbm.at[pl.ds(start, CHUNK)]

    pltpu.async_copy(tab_hbm.at[idx_v.at[0]], rows0, sem0)

    def pair(jj, _):
        j0 = 2 * jj

        @pl.when(jj > 0)
        def _():
            pltpu.make_async_copy(rows1, out_slice(j0 - 1), osem1).wait()
        pltpu.async_copy(tab_hbm.at[idx_v.at[j0 + 1]], rows1, sem1)
        pltpu.make_async_copy(tab_hbm.at[idx_v.at[j0]], rows0, sem0).wait()
        pltpu.async_copy(rows0, out_slice(j0), osem0)

        pltpu.make_async_copy(rows0, out_slice(j0), osem0).wait()
        jn = jnp.minimum(j0 + 2, N_CHUNKS - 1)
        pltpu.async_copy(tab_hbm.at[idx_v.at[jn]], rows0, sem0)
        pltpu.make_async_copy(tab_hbm.at[idx_v.at[j0 + 1]], rows1, sem1).wait()
        pltpu.async_copy(rows1, out_slice(j0 + 1), osem1)
        return 0

    lax.fori_loop(0, N_CHUNKS // 2, pair, 0)
    pltpu.make_async_copy(
        tab_hbm.at[idx_v.at[N_CHUNKS - 1]], rows0, sem0).wait()
    pltpu.make_async_copy(rows1, out_slice(N_CHUNKS - 1), osem1).wait()


def _sc_gather(idx, tab_flat, interpret=False):
    mesh = plsc.VectorSubcoreMesh(
        core_axis_name="c", subcore_axis_name="s",
        num_cores=NC, num_subcores=NS)
    return pl.kernel(
        _sc_gather_body,
        out_type=jax.ShapeDtypeStruct((N_ROWS, EMB), jnp.float32),
        mesh=mesh,
        scratch_types=[
            pltpu.VMEM((N_CHUNKS, CHUNK), jnp.int32),
            pltpu.VMEM((CHUNK, EMB), jnp.float32),
            pltpu.VMEM((CHUNK, EMB), jnp.float32),
            pltpu.SemaphoreType.DMA,
            pltpu.SemaphoreType.DMA,
            pltpu.SemaphoreType.DMA,
            pltpu.SemaphoreType.DMA,
        ],
        compiler_params=pltpu.CompilerParams(use_tc_tiling_on_sc=False),
        interpret=interpret,
    )(idx, tab_flat)


def _mlp_body(se_ref, xd_ref, w1s_ref, w1d_ref, b1_ref, w2_ref, b2_ref,
              w3_ref, b3_ref, ww_ref, bw_ref, mix_ref,
              logit_ref, prob_ref):
    xd = xd_ref[...]
    h = jnp.dot(se_ref[...], w1s_ref[...], preferred_element_type=jnp.float32)
    h += jnp.dot(xd, w1d_ref[...], preferred_element_type=jnp.float32)
    h = jnp.maximum(h + b1_ref[...], 0.0)
    h = jnp.maximum(
        jnp.dot(h, w2_ref[...], preferred_element_type=jnp.float32)
        + b2_ref[...], 0.0)
    deep = jnp.dot(h, w3_ref[...], preferred_element_type=jnp.float32) + b3_ref[...]
    wide = jnp.dot(xd, ww_ref[...], preferred_element_type=jnp.float32) + bw_ref[...]
    e = jnp.exp(mix_ref[...] - jnp.max(mix_ref[...]))
    w = e / jnp.sum(e)
    logit = wide * w[0:1, 0:1] + deep * w[0:1, 1:2]
    logit_ref[...] = logit
    prob_ref[...] = 1.0 / (1.0 + jnp.exp(-logit))


def _mlp(se, xd, w1s, w1d, b1, w2, b2, w3, b3, ww, bw, mix, interpret=False):
    BB = 2048
    grid = (B // BB,)
    const = lambda shape: pl.BlockSpec(shape, lambda i: tuple(0 for _ in shape))
    return pl.pallas_call(
        _mlp_body,
        grid=grid,
        in_specs=[
            pl.BlockSpec((BB, SE_DIM), lambda i: (i, 0)),
            pl.BlockSpec((BB, DENSE), lambda i: (i, 0)),
            const((SE_DIM, 64)),
            const((DENSE, 64)),
            const((1, 64)),
            const((64, 32)),
            const((1, 32)),
            const((32, 1)),
            const((1, 1)),
            const((DENSE, 1)),
            const((1, 1)),
            const((1, 2)),
        ],
        out_specs=[
            pl.BlockSpec((BB, 1), lambda i: (i, 0)),
            pl.BlockSpec((BB, 1), lambda i: (i, 0)),
        ],
        out_shape=[
            jax.ShapeDtypeStruct((B, 1), jnp.float32),
            jax.ShapeDtypeStruct((B, 1), jnp.float32),
        ],
        interpret=interpret,
    )(se, xd, w1s, w1d, b1, w2, b2, w3, b3, ww, bw, mix)


@jax.jit
def kernel(x_sparse, x_dense, tables, W_wide, b_wide, W1, b1, W2, b2, W3, b3, mix):
    t128 = tables.reshape(F_SPARSE * VOCAB * EMB // 128, 128)
    tab_flat = t128.reshape(F_SPARSE * VOCAB, EMB)
    offs = (jnp.arange(F_SPARSE, dtype=jnp.int32) * VOCAB)[None, :]
    idx = (x_sparse.astype(jnp.int32) + offs).reshape(NW, N_CHUNKS, CHUNK)
    se = _sc_gather(idx, tab_flat).reshape(B, SE_DIM)
    logit, prob = _mlp(
        se, x_dense,
        W1[:SE_DIM], W1[SE_DIM:], b1.reshape(1, 64),
        W2, b2.reshape(1, 32), W3, b3.reshape(1, 1),
        W_wide, b_wide.reshape(1, 1), mix.reshape(1, 2))
    return (logit, prob)
